# baseline (speedup 1.0000x reference)
import functools

import jax
import jax.numpy as jnp
import numpy as np
from jax import lax
from jax.experimental import pallas as pl
from jax.experimental.pallas import tpu as pltpu
from jax.experimental.pallas import tpu_sc as plsc

ALPHA_R = 0.65 * 0.1

B = 128
L = 8192
NLANE = 16
NCHUNK = L // NLANE
NW = 32
B_SC = 64
ROWS_PER_W = B_SC // NW
NT = L // 128


def _make_sc_kernel():
    mesh = plsc.VectorSubcoreMesh(core_axis_name="c", subcore_axis_name="s")

    @functools.partial(
        pl.kernel,
        mesh=mesh,
        compiler_params=pltpu.CompilerParams(
            needs_layout_passes=False, use_tc_tiling_on_sc=True
        ),
        out_type=jax.ShapeDtypeStruct((NW, 128), jnp.float32),
        scratch_types=[
            pltpu.VMEM((ROWS_PER_W * 2 * NT, 128), jnp.float32),
            pltpu.VMEM((8 * NT, 128), jnp.int32),
            pltpu.VMEM((NT, 128), jnp.float32),
            pltpu.VMEM((1, 128), jnp.float32),
            pltpu.SemaphoreType.DMA,
        ],
    )
    def sc_body(out_hbm, lab_hbm, nci_hbm, res_hbm, obuf, lbuf, nbuf, rbuf, sem):
        wid = lax.axis_index("s") * 2 + lax.axis_index("c")
        cp_o = pltpu.make_async_copy(
            out_hbm.at[pl.ds(wid * (ROWS_PER_W * 2 * NT), ROWS_PER_W * 2 * NT)],
            obuf,
            sem,
        )
        cp_o.start()
        bt = wid >> 2
        lb0 = (wid * ROWS_PER_W) & 7
        cp_l = pltpu.make_async_copy(
            lab_hbm.at[pl.ds(bt * (8 * NT), 8 * NT)], lbuf, sem
        )
        cp_l.start()
        copies = [cp_l]
        cp_n = pltpu.make_async_copy(nci_hbm, nbuf, sem)
        cp_n.start()
        cp_o.wait()

        lanes = lax.iota(jnp.int32, NLANE)
        zerov = jnp.zeros((NLANE,), jnp.float32)

        cuts = []
        for rr in range(ROWS_PER_W):
            orow0 = rr * (2 * NT)

            def find_cond(state):
                k, last = state
                return jnp.logical_and(last < 0, k >= 0)

            def find_body(state):
                k, _ = state
                tr = orow0 + (k >> 3) * 2
                cs = (k & 7) * NLANE
                c0 = obuf[tr, pl.ds(cs, NLANE)]
                c1 = obuf[tr + 1, pl.ds(cs, NLANE)]
                jv = k * NLANE + lanes
                last = jnp.max(jnp.where(c0 >= c1, jv, -1))
                return k - 1, last

            _, last_zero = lax.while_loop(
                find_cond, find_body, (jnp.int32(NCHUNK - 1), jnp.int32(-1))
            )
            cuts.append(jnp.where(last_zero < 0, L, last_zero))

        for cp in copies:
            cp.wait()
        cp_n.wait()

        def sum_body(k, vas):
            th = k >> 3
            cs = (k & 7) * NLANE
            ncv = nbuf[th, pl.ds(cs, NLANE)]
            jv = k * NLANE + lanes
            out = []
            for rr in range(ROWS_PER_W):
                c1 = obuf[rr * (2 * NT) + th * 2 + 1, pl.ds(cs, NLANE)]
                lab = lbuf[th * 8 + lb0 + rr, pl.ds(cs, NLANE)]
                v = c1 * jnp.where(lab == 1, ncv, ALPHA_R)
                out.append(vas[rr] + jnp.where(jv < cuts[rr], v, 0.0))
            return tuple(out)

        vas = lax.fori_loop(
            0, NCHUNK, sum_body, (zerov,) * ROWS_PER_W, unroll=4
        )
        acc = jnp.float32(0.0)
        for rr in range(ROWS_PER_W):
            acc = acc + jnp.sum(vas[rr])

        for p in range(8):
            rbuf[0, pl.ds(p * NLANE, NLANE)] = jnp.where(
                lanes == 0, jnp.where(p == 0, acc * (1.0 / B), 0.0), 0.0
            )
        pltpu.sync_copy(rbuf, res_hbm.at[pl.ds(wid, 1)])

    return sc_body


_sc_kernel = _make_sc_kernel()


def _tc_body(oref, lref, nref, out_ref):
    ncv = nref[...]
    ti = lax.broadcasted_iota(jnp.int32, (NT, 128), 0)
    ki = lax.broadcasted_iota(jnp.int32, (NT, 128), 1)
    pos = ti * 128 + ki
    ri = lax.broadcasted_iota(jnp.int32, (8, 128), 0)
    li = lax.broadcasted_iota(jnp.int32, (8, 128), 1)
    res = jnp.zeros((8, 128), jnp.float32)
    for s in range(8):
        x3 = oref[pl.ds(s * 128, 128), :].reshape(NT, 2, 128)
        c0 = x3[:, 0, :]
        c1 = x3[:, 1, :]
        lab = lref[0, :, s, :]
        zero = c0 >= c1
        last = jnp.max(jnp.where(zero, pos, -1))
        cut = jnp.where(last < 0, L, last)
        v = c1 * jnp.where(lab == 1, ncv, ALPHA_R)
        psum = jnp.sum(jnp.where(pos < cut, v, 0.0)) * (1.0 / B)
        res = jnp.where(jnp.logical_and(ri == s, li == 0), psum, res)
    out_ref[...] = res


_tc_kernel = pl.pallas_call(
    _tc_body,
    grid=((B - B_SC) // 8,),
    in_specs=[
        pl.BlockSpec((8 * 128, 128), lambda i: (i + B_SC // 8, 0)),
        pl.BlockSpec((1, NT, 8, 128), lambda i: (i + B_SC // 8, 0, 0, 0)),
        pl.BlockSpec((NT, 128), lambda i: (0, 0)),
    ],
    out_specs=pl.BlockSpec((8, 128), lambda i: (i, 0)),
    out_shape=jax.ShapeDtypeStruct((B - B_SC, 128), jnp.float32),
)


@jax.jit
def kernel(output, labels):
    out_v = output.reshape(B, NT, 128, 2).transpose(0, 1, 3, 2).reshape(B * 2 * NT, 128)
    lab4 = labels.reshape(B // 8, 8, NT, 128).transpose(0, 2, 1, 3)
    lab_v = lab4.reshape(B * NT, 128)
    j = np.arange(L, dtype=np.float32)
    nci = jnp.asarray((-3.6 / np.log2(j + 2.0)).reshape(NT, 128))
    sc_partials = _sc_kernel(out_v, lab_v, nci)
    tc_partials = _tc_kernel(out_v, lab4, nci)
    return jnp.sum(sc_partials) + jnp.sum(tc_partials)


# speedup vs baseline: 1.0055x; 1.0055x over previous
import functools

import jax
import jax.numpy as jnp
import numpy as np
from jax import lax
from jax.experimental import pallas as pl
from jax.experimental.pallas import tpu as pltpu
from jax.experimental.pallas import tpu_sc as plsc

ALPHA_R = 0.65 * 0.1

B = 128
L = 8192
NLANE = 16
NCHUNK = L // NLANE
NW = 32
B_SC = 64
ROWS_PER_W = B_SC // NW
NT = L // 128


def _make_sc_kernel():
    mesh = plsc.VectorSubcoreMesh(core_axis_name="c", subcore_axis_name="s")

    @functools.partial(
        pl.kernel,
        mesh=mesh,
        compiler_params=pltpu.CompilerParams(
            needs_layout_passes=False, use_tc_tiling_on_sc=True
        ),
        out_type=jax.ShapeDtypeStruct((NW, 128), jnp.float32),
        scratch_types=[
            pltpu.VMEM((ROWS_PER_W * 2 * NT, 128), jnp.float32),
            pltpu.VMEM((4 * NT, 128), jnp.int32),
            pltpu.VMEM((NT, 128), jnp.float32),
            pltpu.VMEM((1, 128), jnp.float32),
            pltpu.SemaphoreType.DMA,
        ],
    )
    def sc_body(out_hbm, lab_hbm, nci_hbm, res_hbm, obuf, lbuf, nbuf, rbuf, sem):
        wid = lax.axis_index("s") * 2 + lax.axis_index("c")
        cp_o = pltpu.make_async_copy(
            out_hbm.at[pl.ds(wid * (ROWS_PER_W * 2 * NT), ROWS_PER_W * 2 * NT)],
            obuf,
            sem,
        )
        cp_o.start()
        bt = wid >> 2
        s0 = ((wid >> 1) & 1) * 4
        lb0 = (wid & 1) * 2
        copies = []
        for t in range(NT):
            cp = pltpu.make_async_copy(
                lab_hbm.at[pl.ds(bt * (8 * NT) + t * 8 + s0, 4)],
                lbuf.at[pl.ds(t * 4, 4)],
                sem,
            )
            cp.start()
            copies.append(cp)
        cp_n = pltpu.make_async_copy(nci_hbm, nbuf, sem)
        cp_n.start()
        cp_o.wait()

        lanes = lax.iota(jnp.int32, NLANE)
        zerov = jnp.zeros((NLANE,), jnp.float32)

        cuts = []
        for rr in range(ROWS_PER_W):
            orow0 = rr * (2 * NT)

            def find_cond(state):
                k, last = state
                return jnp.logical_and(last < 0, k >= 0)

            def find_body(state):
                k, _ = state
                tr = orow0 + (k >> 3) * 2
                cs = (k & 7) * NLANE
                c0 = obuf[tr, pl.ds(cs, NLANE)]
                c1 = obuf[tr + 1, pl.ds(cs, NLANE)]
                jv = k * NLANE + lanes
                last = jnp.max(jnp.where(c0 >= c1, jv, -1))
                return k - 1, last

            _, last_zero = lax.while_loop(
                find_cond, find_body, (jnp.int32(NCHUNK - 1), jnp.int32(-1))
            )
            cuts.append(jnp.where(last_zero < 0, L, last_zero))

        for cp in copies:
            cp.wait()
        cp_n.wait()

        def sum_body(k, vas):
            th = k >> 3
            cs = (k & 7) * NLANE
            ncv = nbuf[th, pl.ds(cs, NLANE)]
            jv = k * NLANE + lanes
            out = []
            for rr in range(ROWS_PER_W):
                c1 = obuf[rr * (2 * NT) + th * 2 + 1, pl.ds(cs, NLANE)]
                lab = lbuf[th * 4 + lb0 + rr, pl.ds(cs, NLANE)]
                v = c1 * jnp.where(lab == 1, ncv, ALPHA_R)
                out.append(vas[rr] + jnp.where(jv < cuts[rr], v, 0.0))
            return tuple(out)

        vas = lax.fori_loop(
            0, NCHUNK, sum_body, (zerov,) * ROWS_PER_W, unroll=4
        )
        acc = jnp.float32(0.0)
        for rr in range(ROWS_PER_W):
            acc = acc + jnp.sum(vas[rr])

        for p in range(8):
            rbuf[0, pl.ds(p * NLANE, NLANE)] = jnp.where(
                lanes == 0, jnp.where(p == 0, acc * (1.0 / B), 0.0), 0.0
            )
        pltpu.sync_copy(rbuf, res_hbm.at[pl.ds(wid, 1)])

    return sc_body


_sc_kernel = _make_sc_kernel()


def _tc_body(oref, lref, nref, out_ref):
    ncv = nref[...]
    ti = lax.broadcasted_iota(jnp.int32, (NT, 128), 0)
    ki = lax.broadcasted_iota(jnp.int32, (NT, 128), 1)
    pos = ti * 128 + ki
    ri = lax.broadcasted_iota(jnp.int32, (8, 128), 0)
    li = lax.broadcasted_iota(jnp.int32, (8, 128), 1)
    res = jnp.zeros((8, 128), jnp.float32)
    for s in range(8):
        x3 = oref[pl.ds(s * 128, 128), :].reshape(NT, 2, 128)
        c0 = x3[:, 0, :]
        c1 = x3[:, 1, :]
        lab = lref[0, :, s, :]
        zero = c0 >= c1
        last = jnp.max(jnp.where(zero, pos, -1))
        cut = jnp.where(last < 0, L, last)
        v = c1 * jnp.where(lab == 1, ncv, ALPHA_R)
        psum = jnp.sum(jnp.where(pos < cut, v, 0.0)) * (1.0 / B)
        res = jnp.where(jnp.logical_and(ri == s, li == 0), psum, res)
    out_ref[...] = res


_tc_kernel = pl.pallas_call(
    _tc_body,
    grid=((B - B_SC) // 8,),
    in_specs=[
        pl.BlockSpec((8 * 128, 128), lambda i: (i + B_SC // 8, 0)),
        pl.BlockSpec((1, NT, 8, 128), lambda i: (i + B_SC // 8, 0, 0, 0)),
        pl.BlockSpec((NT, 128), lambda i: (0, 0)),
    ],
    out_specs=pl.BlockSpec((8, 128), lambda i: (i, 0)),
    out_shape=jax.ShapeDtypeStruct((B - B_SC, 128), jnp.float32),
)


@jax.jit
def kernel(output, labels):
    out_v = output.reshape(B, NT, 128, 2).transpose(0, 1, 3, 2).reshape(B * 2 * NT, 128)
    lab4 = labels.reshape(B // 8, 8, NT, 128).transpose(0, 2, 1, 3)
    lab_v = lab4.reshape(B * NT, 128)
    j = np.arange(L, dtype=np.float32)
    nci = jnp.asarray((-3.6 / np.log2(j + 2.0)).reshape(NT, 128))
    sc_partials = _sc_kernel(out_v, lab_v, nci)
    tc_partials = _tc_kernel(out_v, lab4, nci)
    return jnp.sum(sc_partials) + jnp.sum(tc_partials)


# speedup vs baseline: 1.0992x; 1.0931x over previous
import functools

import jax
import jax.numpy as jnp
import numpy as np
from jax import lax
from jax.experimental import pallas as pl
from jax.experimental.pallas import tpu as pltpu
from jax.experimental.pallas import tpu_sc as plsc

ALPHA_R = 0.65 * 0.1

B = 128
L = 8192
NLANE = 16
NCHUNK = L // NLANE
NW = 32
B_SC = 64
ROWS_PER_W = B_SC // NW
NT = L // 128


def _make_sc_kernel():
    mesh = plsc.VectorSubcoreMesh(core_axis_name="c", subcore_axis_name="s")

    @functools.partial(
        pl.kernel,
        mesh=mesh,
        compiler_params=pltpu.CompilerParams(
            needs_layout_passes=False, use_tc_tiling_on_sc=True
        ),
        out_type=jax.ShapeDtypeStruct((NW, 128), jnp.float32),
        scratch_types=[
            pltpu.VMEM((ROWS_PER_W * 2 * NT, 128), jnp.float32),
            pltpu.VMEM((4 * NT, 128), jnp.int32),
            pltpu.VMEM((NT, 128), jnp.float32),
            pltpu.VMEM((1, 128), jnp.float32),
            pltpu.SemaphoreType.DMA,
        ],
    )
    def sc_body(out_hbm, lab_hbm, nci_hbm, res_hbm, obuf, lbuf, nbuf, rbuf, sem):
        wid = lax.axis_index("s") * 2 + lax.axis_index("c")
        cp_o = pltpu.make_async_copy(
            out_hbm.at[pl.ds(wid * (ROWS_PER_W * 2 * NT), ROWS_PER_W * 2 * NT)],
            obuf,
            sem,
        )
        cp_o.start()
        bt = wid >> 2
        s0 = ((wid >> 1) & 1) * 4
        lb0 = (wid & 1) * 2
        copies = []
        for t in range(NT):
            cp = pltpu.make_async_copy(
                lab_hbm.at[pl.ds(bt * (8 * NT) + t * 8 + s0, 4)],
                lbuf.at[pl.ds(t * 4, 4)],
                sem,
            )
            cp.start()
            copies.append(cp)
        cp_n = pltpu.make_async_copy(nci_hbm, nbuf, sem)
        cp_n.start()
        cp_o.wait()

        lanes = lax.iota(jnp.int32, NLANE)
        zerov = jnp.zeros((NLANE,), jnp.float32)

        cuts = []
        for rr in range(ROWS_PER_W):
            orow0 = rr * (2 * NT)

            def find_cond(state):
                k, last = state
                return jnp.logical_and(last < 0, k >= 0)

            def find_body(state):
                k, _ = state
                tr = orow0 + (k >> 3) * 2
                cs = (k & 7) * NLANE
                c0 = obuf[tr, pl.ds(cs, NLANE)]
                c1 = obuf[tr + 1, pl.ds(cs, NLANE)]
                jv = k * NLANE + lanes
                last = jnp.max(jnp.where(c0 >= c1, jv, -1))
                return k - 1, last

            _, last_zero = lax.while_loop(
                find_cond, find_body, (jnp.int32(NCHUNK - 1), jnp.int32(-1))
            )
            cuts.append(jnp.where(last_zero < 0, L, last_zero))

        for cp in copies:
            cp.wait()
        cp_n.wait()

        def sum_body(k, vas):
            th = k >> 3
            cs = (k & 7) * NLANE
            ncv = nbuf[th, pl.ds(cs, NLANE)]
            jv = k * NLANE + lanes
            out = []
            for rr in range(ROWS_PER_W):
                c1 = obuf[rr * (2 * NT) + th * 2 + 1, pl.ds(cs, NLANE)]
                lab = lbuf[th * 4 + lb0 + rr, pl.ds(cs, NLANE)]
                v = c1 * jnp.where(lab == 1, ncv, ALPHA_R)
                out.append(vas[rr] + jnp.where(jv < cuts[rr], v, 0.0))
            return tuple(out)

        vas = lax.fori_loop(
            0, NCHUNK, sum_body, (zerov,) * ROWS_PER_W, unroll=4
        )
        acc = jnp.float32(0.0)
        for rr in range(ROWS_PER_W):
            acc = acc + jnp.sum(vas[rr])

        for p in range(8):
            rbuf[0, pl.ds(p * NLANE, NLANE)] = jnp.where(
                lanes == 0, jnp.where(p == 0, acc * (1.0 / B), 0.0), 0.0
            )
        pltpu.sync_copy(rbuf, res_hbm.at[pl.ds(wid, 1)])

    return sc_body


_sc_kernel = _make_sc_kernel()


def _tc_body(oref, lref, nref, out_ref):
    x = oref[...]
    xs = jnp.roll(x, -1, axis=0)
    ri = lax.broadcasted_iota(jnp.int32, (1024, 128), 0)
    ki = lax.broadcasted_iota(jnp.int32, (1024, 128), 1)
    even = (ri & 1) == 0
    pos = ((ri >> 1) & 63) * 128 + ki
    zval = jnp.where(jnp.logical_and(even, x >= xs), pos, -1)
    last8 = jnp.max(zval.reshape(8, 128, 128), axis=(1, 2))
    cut8 = jnp.where(last8 < 0, L, last8)
    pos3 = pos.reshape(8, 128, 128)
    mask3 = jnp.logical_and(
        pos3 < jnp.broadcast_to(cut8.reshape(8, 1, 1), (8, 128, 128)),
        even.reshape(8, 128, 128),
    )
    labE = jnp.broadcast_to(
        jnp.transpose(lref[0], (1, 0, 2)).reshape(8, NT, 1, 128),
        (8, NT, 2, 128),
    ).reshape(8, 128, 128)
    nciE = jnp.broadcast_to(nref[...].reshape(1, 128, 128), (8, 128, 128))
    r1 = jnp.where(labE == 1, nciE, ALPHA_R)
    contrib = jnp.where(mask3, xs.reshape(8, 128, 128) * r1, 0.0)
    psum = jnp.sum(contrib) * (1.0 / B)
    li = lax.broadcasted_iota(jnp.int32, (1, 128), 1)

    @pl.when(pl.program_id(0) == 0)
    def _():
        out_ref[...] = jnp.zeros((1, 128), jnp.float32)

    out_ref[...] = out_ref[...] + jnp.where(li == 0, psum, 0.0)


_tc_kernel = pl.pallas_call(
    _tc_body,
    grid=((B - B_SC) // 8,),
    in_specs=[
        pl.BlockSpec((8 * 128, 128), lambda i: (i + B_SC // 8, 0)),
        pl.BlockSpec((1, NT, 8, 128), lambda i: (i + B_SC // 8, 0, 0, 0)),
        pl.BlockSpec((2 * NT, 128), lambda i: (0, 0)),
    ],
    out_specs=pl.BlockSpec((1, 128), lambda i: (0, 0)),
    out_shape=jax.ShapeDtypeStruct((1, 128), jnp.float32),
)


@jax.jit
def kernel(output, labels):
    out_v = output.reshape(B, NT, 128, 2).transpose(0, 1, 3, 2).reshape(B * 2 * NT, 128)
    lab4 = labels.reshape(B // 8, 8, NT, 128).transpose(0, 2, 1, 3)
    lab_v = lab4.reshape(B * NT, 128)
    j = np.arange(L, dtype=np.float32)
    nci_tab = (-3.6 / np.log2(j + 2.0)).reshape(NT, 128).astype(np.float32)
    nci = jnp.asarray(nci_tab)
    nci2 = jnp.asarray(np.repeat(nci_tab, 2, axis=0))
    sc_partials = _sc_kernel(out_v, lab_v, nci)
    tc_partials = _tc_kernel(out_v, lab4, nci2)
    return jnp.sum(sc_partials) + jnp.sum(tc_partials)


# speedup vs baseline: 1.2280x; 1.1172x over previous
import functools

import jax
import jax.numpy as jnp
import numpy as np
from jax import lax
from jax.experimental import pallas as pl
from jax.experimental.pallas import tpu as pltpu
from jax.experimental.pallas import tpu_sc as plsc

ALPHA_R = 0.65 * 0.1

B = 128
L = 8192
NLANE = 16
NCHUNK = L // NLANE
NW = 32
ROWS_PER_W = B // NW
NT = L // 128


def _make_sc_kernel():
    mesh = plsc.VectorSubcoreMesh(core_axis_name="c", subcore_axis_name="s")

    @functools.partial(
        pl.kernel,
        mesh=mesh,
        compiler_params=pltpu.CompilerParams(
            needs_layout_passes=False, use_tc_tiling_on_sc=True
        ),
        out_type=jax.ShapeDtypeStruct((NW, 128), jnp.float32),
        scratch_types=[
            pltpu.VMEM((ROWS_PER_W * 2 * NT, 128), jnp.float32),
            pltpu.VMEM((ROWS_PER_W * NT, 128), jnp.int32),
            pltpu.VMEM((L // 128, 128), jnp.float32),
            pltpu.VMEM((1, 128), jnp.float32),
            pltpu.SemaphoreType.DMA,
        ],
    )
    def sc_body(out_hbm, lab_hbm, nci_hbm, res_hbm, obuf, lbuf, nbuf, rbuf, sem):
        wid = lax.axis_index("s") * 2 + lax.axis_index("c")
        cp_o = pltpu.make_async_copy(
            out_hbm.at[pl.ds(wid * (ROWS_PER_W * 2 * NT), ROWS_PER_W * 2 * NT)],
            obuf,
            sem,
        )
        cp_o.start()
        bt = wid >> 1
        s0 = (wid & 1) * ROWS_PER_W
        copies = []
        for t in range(NT):
            cp = pltpu.make_async_copy(
                lab_hbm.at[pl.ds(bt * (8 * NT) + t * 8 + s0, ROWS_PER_W)],
                lbuf.at[pl.ds(t * ROWS_PER_W, ROWS_PER_W)],
                sem,
            )
            cp.start()
            copies.append(cp)
        cp_n = pltpu.make_async_copy(nci_hbm, nbuf, sem)
        cp_n.start()
        cp_o.wait()

        lanes = lax.iota(jnp.int32, NLANE)
        zerov = jnp.zeros((NLANE,), jnp.float32)

        cuts = []
        for rr in range(ROWS_PER_W):
            orow0 = rr * (2 * NT)

            def find_cond(state):
                k, last = state
                return jnp.logical_and(last < 0, k >= 0)

            def find_body(state):
                k, _ = state
                tr = orow0 + (k >> 3) * 2
                cs = (k & 7) * NLANE
                c0 = obuf[tr, pl.ds(cs, NLANE)]
                c1 = obuf[tr + 1, pl.ds(cs, NLANE)]
                jv = k * NLANE + lanes
                last = jnp.max(jnp.where(c0 >= c1, jv, -1))
                return k - 1, last

            _, last_zero = lax.while_loop(
                find_cond, find_body, (jnp.int32(NCHUNK - 1), jnp.int32(-1))
            )
            cuts.append(jnp.where(last_zero < 0, L, last_zero))

        for cp in copies:
            cp.wait()
        cp_n.wait()

        def sum_body(k, vas):
            th = k >> 3
            cs = (k & 7) * NLANE
            ncv = nbuf[th, pl.ds(cs, NLANE)]
            jv = k * NLANE + lanes
            out = []
            for rr in range(ROWS_PER_W):
                c1 = obuf[rr * (2 * NT) + th * 2 + 1, pl.ds(cs, NLANE)]
                lab = lbuf[th * ROWS_PER_W + rr, pl.ds(cs, NLANE)]
                v = c1 * jnp.where(lab == 1, ncv, ALPHA_R)
                out.append(vas[rr] + jnp.where(jv < cuts[rr], v, 0.0))
            return tuple(out)

        vas = lax.fori_loop(
            0, NCHUNK, sum_body, (zerov,) * ROWS_PER_W, unroll=2
        )
        acc = jnp.float32(0.0)
        for rr in range(ROWS_PER_W):
            acc = acc + jnp.sum(vas[rr])

        for p in range(8):
            rbuf[0, pl.ds(p * NLANE, NLANE)] = jnp.where(
                lanes == 0, jnp.where(p == 0, acc * (1.0 / B), 0.0), 0.0
            )
        pltpu.sync_copy(rbuf, res_hbm.at[pl.ds(wid, 1)])

    return sc_body


_sc_kernel = _make_sc_kernel()


@jax.jit
def kernel(output, labels):
    out_v = output.reshape(B, NT, 128, 2).transpose(0, 1, 3, 2).reshape(B * 2 * NT, 128)
    lab_v = labels.reshape(B // 8, 8, NT, 128).transpose(0, 2, 1, 3).reshape(B * NT, 128)
    j = np.arange(L, dtype=np.float32)
    nci = jnp.asarray((-3.6 / np.log2(j + 2.0)).reshape(L // 128, 128))
    partials = _sc_kernel(out_v, lab_v, nci)
    return jnp.sum(partials)
